# baseline (speedup 1.0000x reference)
import functools

import jax
import jax.numpy as jnp
from jax import lax
from jax.experimental import pallas as pl
from jax.experimental.pallas import tpu as pltpu
from jax.experimental.pallas import tpu_sc as plsc

N_EMBD = 64
NC = 2
NS = 16
NW = NC * NS
LANES = 16

BATCH = 4096
HIST = 200
TOTAL = BATCH * HIST
B_PER_W = TOTAL // NW
BB = BATCH // NW

_mesh = plsc.VectorSubcoreMesh(core_axis_name="c", subcore_axis_name="s")


@functools.partial(
    pl.kernel,
    mesh=_mesh,
    out_type=jax.ShapeDtypeStruct((HIST, N_EMBD, BATCH), jnp.float32),
    scratch_types=[
        pltpu.VMEM((B_PER_W,), jnp.int32),
        pltpu.VMEM((BB,), jnp.int32),
        pltpu.VMEM((BB,), jnp.int32),
        pltpu.VMEM((BB, N_EMBD), jnp.float32),
        pltpu.VMEM((BB, N_EMBD), jnp.float32),
        pltpu.VMEM((N_EMBD, BB), jnp.float32),
        pltpu.VMEM((N_EMBD, BB), jnp.float32),
        pltpu.SemaphoreType.DMA,
        pltpu.SemaphoreType.DMA,
        pltpu.SemaphoreType.DMA,
        pltpu.SemaphoreType.DMA,
    ],
    compiler_params=pltpu.CompilerParams(
        use_tc_tiling_on_sc=False, needs_layout_passes=False),
)
def _gather_rows(table_hbm, idx_hbm, out_hbm, idx_v, ridx0, ridx1,
                 rows0, rows1, tbuf0, tbuf1, gsem0, gsem1, ssem0, ssem1):
    wid = lax.axis_index("s") * NC + lax.axis_index("c")
    b0 = wid * BB

    pltpu.sync_copy(idx_hbm.at[pl.ds(b0 * HIST, B_PER_W)], idx_v)

    lane = lax.iota(jnp.int32, LANES)

    def compute_idx(h, ridx_v):
        for m in range(BB // LANES):
            bv = (m * LANES + lane) * HIST + h
            ridx_v[pl.ds(m * LANES, LANES)] = plsc.load_gather(idx_v, [bv])

    def gather_start(buf, ridx_v, sem):
        pltpu.async_copy(table_hbm.at[ridx_v], buf, sem)

    def gather_wait(buf, ridx_v, sem):
        pltpu.make_async_copy(table_hbm.at[ridx_v], buf, sem).wait()

    def transpose(rows_v, tbuf_v):
        for m in range(BB // LANES):
            bv = m * LANES + lane
            sl = pl.ds(m * LANES, LANES)
            for c in range(N_EMBD):
                tbuf_v[c, sl] = plsc.load_gather(rows_v, [bv, c + 0 * bv])

    def scatter_start(h, tbuf_v, sem):
        pltpu.async_copy(tbuf_v, out_hbm.at[h, :, pl.ds(b0, BB)], sem)

    def scatter_wait(tbuf_v, sem):
        pltpu.make_async_copy(
            tbuf_v, out_hbm.at[0, :, pl.ds(b0, BB)], sem).wait()

    compute_idx(0, ridx0)
    gather_start(rows0, ridx0, gsem0)

    def body(h, carry):
        even = h % 2 == 0

        def step(ridx_c, rows_c, tbuf_c, gsem_c, ssem_c,
                 ridx_n, rows_n, tbuf_n, gsem_n, ssem_n):
            @pl.when(h + 1 < HIST)
            def _():
                compute_idx(h + 1, ridx_n)
                gather_start(rows_n, ridx_n, gsem_n)

            gather_wait(rows_c, ridx_c, gsem_c)
            @pl.when(h >= 2)
            def _():
                scatter_wait(tbuf_c, ssem_c)
            transpose(rows_c, tbuf_c)
            scatter_start(h, tbuf_c, ssem_c)

        @pl.when(even)
        def _():
            step(ridx0, rows0, tbuf0, gsem0, ssem0,
                 ridx1, rows1, tbuf1, gsem1, ssem1)

        @pl.when(jnp.logical_not(even))
        def _():
            step(ridx1, rows1, tbuf1, gsem1, ssem1,
                 ridx0, rows0, tbuf0, gsem0, ssem0)

        return carry

    lax.fori_loop(0, HIST, body, 0)

    scatter_wait(tbuf0, ssem0)
    scatter_wait(tbuf1, ssem1)


def kernel(x, food_vectors):
    flat = x.reshape(TOTAL)
    out_t = _gather_rows(food_vectors, flat)
    return jnp.transpose(out_t, (2, 0, 1))


# speedup vs baseline: 2.2003x; 2.2003x over previous
import functools

import jax
import jax.numpy as jnp
from jax import lax
from jax.experimental import pallas as pl
from jax.experimental.pallas import tpu as pltpu
from jax.experimental.pallas import tpu_sc as plsc

N_EMBD = 64
PAD = 128
NC = 2
NS = 16
NW = NC * NS

BATCH = 4096
HIST = 200
TOTAL = BATCH * HIST
B_PER_W = TOTAL // NW
ROWS_PER_W = BATCH // NW
KB = 4
CHUNK = KB * HIST
N_CHUNKS = ROWS_PER_W // KB

_mesh = plsc.VectorSubcoreMesh(core_axis_name="c", subcore_axis_name="s")


@functools.partial(
    pl.kernel,
    mesh=_mesh,
    out_type=jax.ShapeDtypeStruct((BATCH, HIST, PAD), jnp.float32),
    scratch_types=[
        pltpu.VMEM((B_PER_W,), jnp.int32),
        pltpu.VMEM((CHUNK, N_EMBD), jnp.float32),
        pltpu.VMEM((CHUNK, N_EMBD), jnp.float32),
        pltpu.SemaphoreType.DMA,
        pltpu.SemaphoreType.DMA,
        pltpu.SemaphoreType.DMA,
        pltpu.SemaphoreType.DMA,
    ],
    compiler_params=pltpu.CompilerParams(use_tc_tiling_on_sc=False),
)
def _gather_rows(table_hbm, idx_hbm, out_hbm, idx_v, rows0, rows1,
                 gsem0, gsem1, ssem0, ssem1):
    wid = lax.axis_index("s") * NC + lax.axis_index("c")
    base = wid * B_PER_W
    row_base = wid * ROWS_PER_W

    pltpu.sync_copy(idx_hbm.at[pl.ds(base, B_PER_W)], idx_v)

    def gather_start(i, buf, sem):
        return pltpu.async_copy(
            table_hbm.at[idx_v.at[pl.ds(i * CHUNK, CHUNK)]], buf, sem)

    def scatter_start(i, buf, sem):
        b0 = row_base + i * KB
        for j in range(KB):
            pltpu.async_copy(
                buf.at[pl.ds(j * HIST, HIST)],
                out_hbm.at[b0 + j, :, pl.ds(0, N_EMBD)],
                sem)

    def scatter_wait(buf, sem):
        for j in range(KB):
            pltpu.make_async_copy(
                buf.at[pl.ds(j * HIST, HIST)],
                out_hbm.at[0, :, pl.ds(0, N_EMBD)],
                sem).wait()

    gather_start(0, rows0, gsem0)

    def body(i, carry):
        even = i % 2 == 0

        def step(rows_cur, rows_nxt, gsem_cur, gsem_nxt, ssem_cur, ssem_nxt):
            @pl.when(i + 1 < N_CHUNKS)
            def _():
                @pl.when(i >= 1)
                def _():
                    scatter_wait(rows_nxt, ssem_nxt)
                gather_start(i + 1, rows_nxt, gsem_nxt)

            pltpu.make_async_copy(
                table_hbm.at[idx_v.at[pl.ds(0, CHUNK)]], rows_cur, gsem_cur
            ).wait()
            scatter_start(i, rows_cur, ssem_cur)

        @pl.when(even)
        def _():
            step(rows0, rows1, gsem0, gsem1, ssem0, ssem1)

        @pl.when(jnp.logical_not(even))
        def _():
            step(rows1, rows0, gsem1, gsem0, ssem1, ssem0)

        return carry

    lax.fori_loop(0, N_CHUNKS, body, 0)

    scatter_wait(rows0, ssem0)
    scatter_wait(rows1, ssem1)


def kernel(x, food_vectors):
    flat = x.reshape(TOTAL)
    out = _gather_rows(food_vectors, flat)
    return out[:, :, :N_EMBD]
